# baseline (speedup 1.0000x reference)
import jax
import jax.numpy as jnp
from jax import lax
from jax.experimental import pallas as pl
from jax.experimental.pallas import tpu as pltpu
from jax.experimental.pallas import tpu_sc as plsc

_EMBED = 64
_B = 4096
_L = 200
_TOTAL = _B * _L
_NC = 2
_NS = 16
_NW = _NC * _NS
_PER_W = _TOTAL // _NW
_KB = 4
_CH = _KB * _L
_NCHUNK = _PER_W // _CH
_NV = _EMBED // 16


def _emb_body(idx_hbm, table_hbm, pos_hbm, out_hbm, idx_v, rows_v, pos_v, sem):
    wid = lax.axis_index("s") * _NC + lax.axis_index("c")
    pltpu.sync_copy(pos_hbm, pos_v)
    w_base = wid * _PER_W

    def chunk_body(c, carry):
        base = w_base + c * _CH
        pltpu.sync_copy(idx_hbm.at[pl.ds(base, _CH)], idx_v)
        pltpu.async_copy(table_hbm.at[idx_v], rows_v, sem).wait()

        def row_body(l, carry2):
            for kb in range(_KB):
                r = kb * _L + l
                for e in range(_NV):
                    s = pl.ds(e * 16, 16)
                    rows_v[r, s] = rows_v[r, s] + pos_v[l, s]
            return carry2

        lax.fori_loop(0, _L, row_body, 0)
        pltpu.sync_copy(rows_v, out_hbm.at[pl.ds(base, _CH)])
        return carry

    lax.fori_loop(0, _NCHUNK, chunk_body, 0)


def kernel(inputs, word_table, pos_table):
    idx = inputs.reshape(-1).astype(jnp.int32)
    pos = pos_table[:_L]
    mesh = plsc.VectorSubcoreMesh(core_axis_name="c", subcore_axis_name="s")
    out = pl.kernel(
        _emb_body,
        out_type=jax.ShapeDtypeStruct((_TOTAL, _EMBED), jnp.float32),
        mesh=mesh,
        compiler_params=pltpu.CompilerParams(use_tc_tiling_on_sc=False),
        scratch_types=[
            pltpu.VMEM((_CH,), jnp.int32),
            pltpu.VMEM((_CH, _EMBED), jnp.float32),
            pltpu.VMEM((_L, _EMBED), jnp.float32),
            pltpu.SemaphoreType.DMA,
        ],
    )(idx, word_table, pos)
    return out.reshape(_B, _L, _EMBED)
